# Initial kernel scaffold; baseline (speedup 1.0000x reference)
import jax
import jax.numpy as jnp
from jax.experimental import pallas as pl


def kernel(x, W1, W2, Wg, G):
    raise NotImplementedError("write your pallas kernel here")


# speedup vs baseline: 1.1331x; 1.1331x over previous
import functools

import jax
import jax.numpy as jnp
from jax import lax
from jax.experimental import pallas as pl
from jax.experimental.pallas import tpu as pltpu

_N_TOK = 8192
_D = 2048
_SQRT_K = 8
_NUM_EXPERTS = 64
_TOP_K = 8
_GATE_RANK = 16

_BLK = 256
_HI = lax.Precision.HIGHEST


def _router_body(x_ref, wct_ref, gt_ref, idx_ref, wts_ref, sel_ref):
    x = x_ref[...].astype(jnp.bfloat16)
    y = jnp.dot(x, wct_ref[...].astype(jnp.bfloat16),
                preferred_element_type=jnp.float32)
    s1 = y[:, 0:_SQRT_K]
    s2 = y[:, _SQRT_K:2 * _SQRT_K]
    qg = y[:, 2 * _SQRT_K:]

    col = lax.broadcasted_iota(jnp.int32, (_SQRT_K, _NUM_EXPERTS), 1)
    row = lax.broadcasted_iota(jnp.int32, (_SQRT_K, _NUM_EXPERTS), 0)
    a = ((col // _SQRT_K) == row).astype(jnp.float32)
    b = ((col % _SQRT_K) == row).astype(jnp.float32)
    sel = (jnp.dot(s1, a, precision=_HI, preferred_element_type=jnp.float32)
           + jnp.dot(s2, b, precision=_HI, preferred_element_type=jnp.float32))
    gate = jnp.dot(qg.astype(jnp.bfloat16), gt_ref[...].astype(jnp.bfloat16),
                   preferred_element_type=jnp.float32)
    tot = sel + gate
    sel_ref[...] = sel

    lanes = lax.broadcasted_iota(jnp.int32, (_BLK, _NUM_EXPERTS), 1)
    w = sel
    idx_cols = []
    comb_cols = []
    for _ in range(_TOP_K):
        m = jnp.max(w, axis=1, keepdims=True)
        cand = jnp.where(w == m, lanes, _NUM_EXPERTS)
        idx = jnp.min(cand, axis=1, keepdims=True)
        oneh = lanes == idx
        tv = jnp.sum(jnp.where(oneh, tot, 0.0), axis=1, keepdims=True)
        idx_cols.append(idx)
        comb_cols.append(tv)
        w = jnp.where(oneh, -jnp.inf, w)

    idx_ref[...] = jnp.concatenate(idx_cols, axis=1)
    comb = jnp.concatenate(comb_cols, axis=1)
    mx = jnp.max(comb, axis=1, keepdims=True)
    e = jnp.exp(comb - mx)
    wts_ref[...] = e / jnp.sum(e, axis=1, keepdims=True)


@jax.jit
def kernel(x, W1, W2, Wg, G):
    wct = jnp.concatenate([W1, W2, Wg], axis=0).T
    gt = G.T
    grid = (_N_TOK // _BLK,)
    idx, wts, sel = pl.pallas_call(
        _router_body,
        grid=grid,
        in_specs=[
            pl.BlockSpec((_BLK, _D), lambda i: (i, 0)),
            pl.BlockSpec((_D, 2 * _SQRT_K + _GATE_RANK), lambda i: (0, 0)),
            pl.BlockSpec((_GATE_RANK, _NUM_EXPERTS), lambda i: (0, 0)),
        ],
        out_specs=[
            pl.BlockSpec((_BLK, _TOP_K), lambda i: (i, 0)),
            pl.BlockSpec((_BLK, _TOP_K), lambda i: (i, 0)),
            pl.BlockSpec((_BLK, _NUM_EXPERTS), lambda i: (i, 0)),
        ],
        out_shape=[
            jax.ShapeDtypeStruct((_N_TOK, _TOP_K), jnp.int32),
            jax.ShapeDtypeStruct((_N_TOK, _TOP_K), jnp.float32),
            jax.ShapeDtypeStruct((_N_TOK, _NUM_EXPERTS), jnp.float32),
        ],
        compiler_params=pltpu.CompilerParams(
            dimension_semantics=("arbitrary",),
        ),
    )(x, wct, gt)
    return idx, wts, sel


# speedup vs baseline: 1.3642x; 1.2040x over previous
import functools

import jax
import jax.numpy as jnp
from jax import lax
from jax.experimental import pallas as pl
from jax.experimental.pallas import tpu as pltpu

_N_TOK = 8192
_D = 2048
_SQRT_K = 8
_NUM_EXPERTS = 64
_TOP_K = 8
_GATE_RANK = 16

_BLK = 256
_HI = lax.Precision.HIGHEST


def _router_body(x_ref, wct_ref, gt_ref, idx_ref, wts_ref, sel_ref):
    x = x_ref[...].astype(jnp.bfloat16)
    y = jnp.dot(x, wct_ref[...].astype(jnp.bfloat16),
                preferred_element_type=jnp.float32)
    s1 = y[:, 0:_SQRT_K]
    s2 = y[:, _SQRT_K:2 * _SQRT_K]
    qg = y[:, 2 * _SQRT_K:]

    col = lax.broadcasted_iota(jnp.int32, (_SQRT_K, _NUM_EXPERTS), 1)
    row = lax.broadcasted_iota(jnp.int32, (_SQRT_K, _NUM_EXPERTS), 0)
    a = ((col // _SQRT_K) == row).astype(jnp.float32)
    b = ((col % _SQRT_K) == row).astype(jnp.float32)
    sel = (jnp.dot(s1, a, precision=_HI, preferred_element_type=jnp.float32)
           + jnp.dot(s2, b, precision=_HI, preferred_element_type=jnp.float32))
    gate = jnp.dot(qg.astype(jnp.bfloat16), gt_ref[...].astype(jnp.bfloat16),
                   preferred_element_type=jnp.float32)
    tot = sel + gate
    sel_ref[...] = sel

    lanes = lax.broadcasted_iota(
        jnp.int32, (_BLK, _NUM_EXPERTS), 1).astype(jnp.float32)
    w = sel
    idx_cols = []
    comb_cols = []
    for _ in range(_TOP_K):
        m = jnp.max(w, axis=1, keepdims=True)
        cand = jnp.where(w == m, lanes, float(_NUM_EXPERTS))
        idx = jnp.min(cand, axis=1, keepdims=True)
        oneh = lanes == idx
        tv = jnp.sum(jnp.where(oneh, tot, 0.0), axis=1, keepdims=True)
        idx_cols.append(idx)
        comb_cols.append(tv)
        w = jnp.where(oneh, -jnp.inf, w)

    idx_ref[...] = jnp.concatenate(idx_cols, axis=1).astype(jnp.int32)
    comb = jnp.concatenate(comb_cols, axis=1)
    mx = jnp.max(comb, axis=1, keepdims=True)
    e = jnp.exp(comb - mx)
    wts_ref[...] = e / jnp.sum(e, axis=1, keepdims=True)


@jax.jit
def kernel(x, W1, W2, Wg, G):
    wct = jnp.concatenate([W1, W2, Wg], axis=0).T
    gt = G.T
    grid = (_N_TOK // _BLK,)
    idx, wts, sel = pl.pallas_call(
        _router_body,
        grid=grid,
        in_specs=[
            pl.BlockSpec((_BLK, _D), lambda i: (i, 0)),
            pl.BlockSpec((_D, 2 * _SQRT_K + _GATE_RANK), lambda i: (0, 0)),
            pl.BlockSpec((_GATE_RANK, _NUM_EXPERTS), lambda i: (0, 0)),
        ],
        out_specs=[
            pl.BlockSpec((_BLK, _TOP_K), lambda i: (i, 0)),
            pl.BlockSpec((_BLK, _TOP_K), lambda i: (i, 0)),
            pl.BlockSpec((_BLK, _NUM_EXPERTS), lambda i: (i, 0)),
        ],
        out_shape=[
            jax.ShapeDtypeStruct((_N_TOK, _TOP_K), jnp.int32),
            jax.ShapeDtypeStruct((_N_TOK, _TOP_K), jnp.float32),
            jax.ShapeDtypeStruct((_N_TOK, _NUM_EXPERTS), jnp.float32),
        ],
        compiler_params=pltpu.CompilerParams(
            dimension_semantics=("arbitrary",),
        ),
    )(x, wct, gt)
    return idx, wts, sel
